# Initial kernel scaffold; baseline (speedup 1.0000x reference)
import jax
import jax.numpy as jnp
from jax.experimental import pallas as pl


def kernel(pred, target):
    raise NotImplementedError("write your pallas kernel here")


# speedup vs baseline: 2.5790x; 2.5790x over previous
import jax
import jax.numpy as jnp
from jax.experimental import pallas as pl
from jax.experimental.pallas import tpu as pltpu

_SMOOTHING = 0.1
_CONFIDENCE = 1.0 - _SMOOTHING


def _body(tgt_ref, pred_ref, out_ref, *, n_rows, n_classes):
    i = pl.program_id(0)
    x = pred_ref[...]
    t = tgt_ref[...]
    m = jnp.max(x, axis=1, keepdims=True)
    s = jnp.sum(jnp.exp(x - m), axis=1, keepdims=True)
    sum_pred = jnp.sum(x, axis=1, keepdims=True)
    cols = jax.lax.broadcasted_iota(jnp.int32, x.shape, 1)
    pt = jnp.sum(jnp.where(cols == t, x, 0.0), axis=1, keepdims=True)
    lse = m + jnp.log(s)
    eps = _SMOOTHING / (n_classes - 1)
    row_loss = -(eps * (sum_pred - n_classes * lse)
                 + (_CONFIDENCE - eps) * (pt - lse))
    part = jnp.sum(row_loss) * (1.0 / n_rows)

    @pl.when(i == 0)
    def _():
        out_ref[0, 0] = 0.0

    out_ref[0, 0] += part


def kernel(pred, target):
    n_rows, n_classes = pred.shape
    r = 32
    t2 = target.astype(jnp.int32).reshape(n_rows, 1)
    body = lambda a, b, o: _body(a, b, o, n_rows=n_rows, n_classes=n_classes)
    out = pl.pallas_call(
        body,
        grid=(n_rows // r,),
        in_specs=[
            pl.BlockSpec((r, 1), lambda i: (i, 0)),
            pl.BlockSpec((r, n_classes), lambda i: (i, 0)),
        ],
        out_specs=pl.BlockSpec((1, 1), lambda i: (0, 0),
                               memory_space=pltpu.SMEM),
        out_shape=jax.ShapeDtypeStruct((1, 1), jnp.float32),
    )(t2, pred)
    return out[0, 0]
